# baseline (speedup 1.0000x reference)
import functools

import jax
import jax.numpy as jnp
from jax import lax
from jax.experimental import pallas as pl
from jax.experimental.pallas import tpu as pltpu
from jax.experimental.pallas import tpu_sc as plsc


NUM_CORES = 2
NUM_SUBCORES = 16
NUM_WORKERS = NUM_CORES * NUM_SUBCORES

GATHER_CHUNK = 128
MM_BLOCK_ROWS = 2048


def _sc_gather(table, idx_flat):
    n = idx_flat.shape[0]
    d = table.shape[1]
    b_per_w = n // NUM_WORKERS
    nchunk = b_per_w // GATHER_CHUNK
    assert n % NUM_WORKERS == 0 and b_per_w % GATHER_CHUNK == 0
    assert nchunk % 2 == 0

    mesh = plsc.VectorSubcoreMesh(core_axis_name="c", subcore_axis_name="s")

    @functools.partial(
        pl.kernel,
        out_type=jax.ShapeDtypeStruct((n, d), table.dtype),
        mesh=mesh,
        scratch_types=[
            pltpu.VMEM((b_per_w,), jnp.int32),
            pltpu.VMEM((GATHER_CHUNK, d), jnp.float32),
            pltpu.VMEM((GATHER_CHUNK, d), jnp.float32),
            pltpu.SemaphoreType.DMA,
            pltpu.SemaphoreType.DMA,
        ],
    )
    def gather_kernel(table_hbm, idx_hbm, out_hbm, idx_v, buf0, buf1, sem0, sem1):
        wid = lax.axis_index("s") * NUM_CORES + lax.axis_index("c")
        base = wid * b_per_w

        pltpu.sync_copy(idx_hbm.at[pl.ds(base, b_per_w)], idx_v)

        def start_gather(c, buf, sem):
            off = pl.multiple_of(c * GATHER_CHUNK, GATHER_CHUNK)
            for g in range(GATHER_CHUNK // 16):
                vec = idx_v[pl.ds(off + g * 16, 16)]
                for t in range(16):
                    pltpu.async_copy(
                        table_hbm.at[pl.ds(vec[t], 1)],
                        buf.at[pl.ds(g * 16 + t, 1)],
                        sem,
                    )

        def wait_gather(c, buf, sem):
            for j in range(GATHER_CHUNK):
                pltpu.make_async_copy(
                    table_hbm.at[pl.ds(0, 1)], buf.at[pl.ds(j, 1)], sem
                ).wait()

        def write_out(c, buf):
            row = base + c * GATHER_CHUNK
            pltpu.sync_copy(buf, out_hbm.at[pl.ds(row, GATHER_CHUNK)])

        start_gather(0, buf0, sem0)

        @pl.loop(0, nchunk // 2)
        def _(i):
            c0 = i * 2
            start_gather(c0 + 1, buf1, sem1)
            wait_gather(c0, buf0, sem0)
            write_out(c0, buf0)

            @pl.when(c0 + 2 < nchunk)
            def _():
                start_gather(c0 + 2, buf0, sem0)

            wait_gather(c0 + 1, buf1, sem1)
            write_out(c0 + 1, buf1)

    return gather_kernel(table, idx_flat)


def _mm_body(emb_ref, wt_ref, out_ref):
    out_ref[...] = jnp.dot(
        emb_ref[...],
        wt_ref[...],
        preferred_element_type=jnp.float32,
        precision=lax.Precision.HIGHEST,
    )


def _tc_matmul(emb, wt):
    n, k = emb.shape
    m = wt.shape[1]
    grid = (n // MM_BLOCK_ROWS,)
    return pl.pallas_call(
        _mm_body,
        grid=grid,
        in_specs=[
            pl.BlockSpec((MM_BLOCK_ROWS, k), lambda i: (i, 0)),
            pl.BlockSpec((k, m), lambda i: (0, 0)),
        ],
        out_specs=pl.BlockSpec((MM_BLOCK_ROWS, m), lambda i: (i, 0)),
        out_shape=jax.ShapeDtypeStruct((n, m), jnp.float32),
    )(emb, wt)


def kernel(x, table, W):
    b, h = x.shape
    idx_flat = x.reshape(b * h).astype(jnp.int32)
    emb = _sc_gather(table, idx_flat)
    out_flat = _tc_matmul(emb, W.T)
    return out_flat.reshape(b, h, W.shape[0])


# speedup vs baseline: 1.3226x; 1.3226x over previous
import functools

import jax
import jax.numpy as jnp
from jax import lax
from jax.experimental import pallas as pl
from jax.experimental.pallas import tpu as pltpu
from jax.experimental.pallas import tpu_sc as plsc


NUM_CORES = 2
NUM_SUBCORES = 16
NUM_WORKERS = NUM_CORES * NUM_SUBCORES

GATHER_CHUNK = 128
MM_BLOCK_ROWS = 2048


def _sc_gather(table, idx_flat):
    n = idx_flat.shape[0]
    d = table.shape[1]
    b_per_w = n // NUM_WORKERS
    nchunk = b_per_w // GATHER_CHUNK
    assert n % NUM_WORKERS == 0 and b_per_w % GATHER_CHUNK == 0
    assert nchunk % 2 == 0

    mesh = plsc.VectorSubcoreMesh(core_axis_name="c", subcore_axis_name="s")

    @functools.partial(
        pl.kernel,
        out_type=jax.ShapeDtypeStruct((n, d), table.dtype),
        mesh=mesh,
        scratch_types=[
            pltpu.VMEM((b_per_w,), jnp.int32),
            pltpu.VMEM((GATHER_CHUNK, d), jnp.float32),
            pltpu.VMEM((GATHER_CHUNK, d), jnp.float32),
            pltpu.SemaphoreType.DMA,
            pltpu.SemaphoreType.DMA,
        ],
    )
    def gather_kernel(table_hbm, idx_hbm, out_hbm, idx_v, buf0, buf1, sem0, sem1):
        wid = lax.axis_index("s") * NUM_CORES + lax.axis_index("c")
        base = wid * b_per_w

        pltpu.sync_copy(idx_hbm.at[pl.ds(base, b_per_w)], idx_v)

        def start_gather(c, buf, sem):
            off = pl.multiple_of(c * GATHER_CHUNK, GATHER_CHUNK)
            for g in range(GATHER_CHUNK // 16):
                vec = idx_v[pl.ds(off + g * 16, 16)]
                for t in range(16):
                    pltpu.async_copy(
                        table_hbm.at[pl.ds(vec[t], 1)],
                        buf.at[pl.ds(g * 16 + t, 1)],
                        sem,
                    )

        def wait_gather(c, buf, sem):
            for j in range(GATHER_CHUNK):
                pltpu.make_async_copy(
                    table_hbm.at[pl.ds(0, 1)], buf.at[pl.ds(j, 1)], sem
                ).wait()

        def write_out(c, buf):
            row = base + c * GATHER_CHUNK
            pltpu.sync_copy(buf, out_hbm.at[pl.ds(row, GATHER_CHUNK)])

        start_gather(0, buf0, sem0)

        @pl.loop(0, nchunk // 2)
        def _(i):
            c0 = i * 2
            start_gather(c0 + 1, buf1, sem1)
            wait_gather(c0, buf0, sem0)
            write_out(c0, buf0)

            @pl.when(c0 + 2 < nchunk)
            def _():
                start_gather(c0 + 2, buf0, sem0)

            wait_gather(c0 + 1, buf1, sem1)
            write_out(c0 + 1, buf1)

    return gather_kernel(table, idx_flat)


MM_BLOCK_BATCH = 256


def _mm_body(emb_ref, w_ref, out_ref):
    bb = out_ref.shape[0]
    h = out_ref.shape[1]
    m = out_ref.shape[2]
    acc = lax.dot_general(
        emb_ref[...],
        w_ref[...],
        (((1,), (1,)), ((), ())),
        preferred_element_type=jnp.float32,
    )
    out_ref[...] = acc.reshape(bb, h, m)


def _tc_matmul(emb, W, b, h):
    n, k = emb.shape
    m = W.shape[0]
    bb = MM_BLOCK_BATCH
    grid = (b // bb,)
    return pl.pallas_call(
        _mm_body,
        grid=grid,
        in_specs=[
            pl.BlockSpec((bb * h, k), lambda i: (i, 0)),
            pl.BlockSpec((m, k), lambda i: (0, 0)),
        ],
        out_specs=pl.BlockSpec((bb, h, m), lambda i: (i, 0, 0)),
        out_shape=jax.ShapeDtypeStruct((b, h, m), jnp.float32),
    )(emb, W)


def kernel(x, table, W):
    b, h = x.shape
    idx_flat = x.reshape(b * h).astype(jnp.int32)
    emb = _sc_gather(table, idx_flat)
    return _tc_matmul(emb, W, b, h)
